# Initial kernel scaffold; baseline (speedup 1.0000x reference)
import jax
import jax.numpy as jnp
from jax.experimental import pallas as pl


def kernel(x, token_table, pos_table):
    raise NotImplementedError("write your pallas kernel here")


# speedup vs baseline: 1.8420x; 1.8420x over previous
import functools

import jax
import jax.numpy as jnp
from jax import lax
from jax.experimental import pallas as pl
from jax.experimental.pallas import tpu as pltpu
from jax.experimental.pallas import tpu_sc as plsc

NC = 2
NS = 16
NW = NC * NS
LANES = 16
CHUNK = 128


def _make_kernel(N, V, L, D):
    rows_per_w = N // NW
    n_chunks = rows_per_w // CHUNK
    mesh = plsc.VectorSubcoreMesh(
        core_axis_name="c", subcore_axis_name="s", num_cores=NC, num_subcores=NS
    )

    @functools.partial(
        pl.kernel,
        out_type=jax.ShapeDtypeStruct((N, D), jnp.float32),
        mesh=mesh,
        scratch_types=[
            pltpu.VMEM((2 * L, D), jnp.float32),
            pltpu.VMEM((CHUNK,), jnp.int32),
            pltpu.VMEM((CHUNK, D), jnp.float32),
            pltpu.SemaphoreType.DMA,
        ],
    )
    def k(x_hbm, tok_hbm, pos_hbm, out_hbm, pos2_v, idx_v, rows_v, sem):
        wid = lax.axis_index("s") * NC + lax.axis_index("c")
        base = wid * rows_per_w
        pltpu.sync_copy(pos_hbm, pos2_v.at[pl.ds(0, L)])
        pltpu.sync_copy(pos_hbm, pos2_v.at[pl.ds(L, L)])

        def chunk_body(c, carry):
            g = base + c * CHUNK
            pltpu.sync_copy(x_hbm.at[pl.ds(g, CHUNK)], idx_v)
            pltpu.async_copy(tok_hbm.at[idx_v], rows_v, sem).wait()
            p0 = lax.rem(c * CHUNK, L)

            def row_body(r, _):
                pr = p0 + r
                for j in range(D // LANES):
                    s = pl.ds(j * LANES, LANES)
                    rows_v[r, s] = rows_v[r, s] + pos2_v[pr, s]
                return _

            lax.fori_loop(0, CHUNK, row_body, 0)
            pltpu.sync_copy(rows_v, out_hbm.at[pl.ds(g, CHUNK)])
            return carry

        lax.fori_loop(0, n_chunks, chunk_body, 0)

    return k


def kernel(x, token_table, pos_table):
    B, L = x.shape
    V, D = token_table.shape
    N = B * L
    x_flat = x.reshape(N).astype(jnp.int32)
    out = _make_kernel(N, V, L, D)(x_flat, token_table, pos_table)
    return out.reshape(B, L, D)


# speedup vs baseline: 2.2011x; 1.1950x over previous
import functools

import jax
import jax.numpy as jnp
from jax import lax
from jax.experimental import pallas as pl
from jax.experimental.pallas import tpu as pltpu
from jax.experimental.pallas import tpu_sc as plsc

NC = 2
NS = 16
NW = NC * NS
LANES = 16
CHUNK = 128


def _make_kernel(N, V, L, D):
    rows_per_w = N // NW
    n_chunks = rows_per_w // CHUNK
    mesh = plsc.VectorSubcoreMesh(
        core_axis_name="c", subcore_axis_name="s", num_cores=NC, num_subcores=NS
    )

    @functools.partial(
        pl.kernel,
        out_type=jax.ShapeDtypeStruct((N, D), jnp.float32),
        mesh=mesh,
        scratch_types=[
            pltpu.VMEM((L + CHUNK, D), jnp.float32),
            pltpu.VMEM((n_chunks, CHUNK), jnp.int32),
            pltpu.VMEM((2, CHUNK, D), jnp.float32),
            pltpu.SemaphoreType.DMA((2,)),
            pltpu.SemaphoreType.DMA((2,)),
        ],
    )
    def k(x2_hbm, tok_hbm, pos_hbm, out_hbm, pos2_v, idxs_v, rows_v, semg, sems):
        wid = lax.axis_index("s") * NC + lax.axis_index("c")
        base = wid * rows_per_w
        pltpu.sync_copy(pos_hbm, pos2_v.at[pl.ds(0, L)])
        pltpu.sync_copy(pos_hbm.at[pl.ds(0, CHUNK)], pos2_v.at[pl.ds(L, CHUNK)])
        pltpu.sync_copy(x2_hbm.at[pl.ds(wid * n_chunks, n_chunks)], idxs_v)
        pltpu.async_copy(tok_hbm.at[idxs_v.at[0]], rows_v.at[0], semg.at[0])

        def chunk_body(c, carry):
            p = lax.rem(c, 2)
            q = 1 - p

            @pl.when(c + 1 < n_chunks)
            def _prefetch():
                @pl.when(c >= 1)
                def _drain_store():
                    pltpu.make_async_copy(
                        rows_v.at[q], out_hbm.at[pl.ds(base, CHUNK)], sems.at[q]
                    ).wait()

                pltpu.async_copy(
                    tok_hbm.at[idxs_v.at[c + 1]], rows_v.at[q], semg.at[q]
                )

            pltpu.make_async_copy(
                tok_hbm.at[pl.ds(0, CHUNK)], rows_v.at[p], semg.at[p]
            ).wait()

            p0 = lax.rem(c * CHUNK, L)

            def row_body(r, _):
                pr = p0 + r
                for j in range(D // LANES):
                    s = pl.ds(j * LANES, LANES)
                    rows_v[p, r, s] = rows_v[p, r, s] + pos2_v[pr, s]
                return _

            lax.fori_loop(0, CHUNK, row_body, 0)
            pltpu.async_copy(
                rows_v.at[p], out_hbm.at[pl.ds(base + c * CHUNK, CHUNK)], sems.at[p]
            )
            return carry

        lax.fori_loop(0, n_chunks, chunk_body, 0)
        for p in range(2):
            pltpu.make_async_copy(
                rows_v.at[p], out_hbm.at[pl.ds(base, CHUNK)], sems.at[p]
            ).wait()

    return k


def kernel(x, token_table, pos_table):
    B, L = x.shape
    V, D = token_table.shape
    N = B * L
    x2 = x.reshape(N // CHUNK, CHUNK).astype(jnp.int32)
    out = _make_kernel(N, V, L, D)(x2, token_table, pos_table)
    return out.reshape(B, L, D)
